# baseline (speedup 1.0000x reference)
import functools

import jax
import jax.numpy as jnp
from jax import lax
from jax.experimental import pallas as pl
from jax.experimental.pallas import tpu as pltpu
from jax.experimental.pallas import tpu_sc as plsc

M_LAYERS = 10
D = 128
N_NODES = 10000
N_EDGES = 320000

NC = 2
NS = 16
NW = NC * NS
EPW = N_EDGES // NW
CHUNK = 16
NSETS = 6
RC = NSETS * CHUNK
N_PAD = 10240
ROWS_PER_SUB = N_PAD // NS

NODE_BLK = 2000
EDGE_BLK = 2000


def _proj_body(x_ref, ws_ref, wd_ref, be_ref, ps_ref, pd_ref):
    x = x_ref[...]
    ps_ref[...] = jnp.dot(x, ws_ref[...], preferred_element_type=jnp.float32)
    pd_ref[...] = (jnp.dot(x, wd_ref[...], preferred_element_type=jnp.float32)
                   + be_ref[...])


def _proj(x, ws, wd, be):
    return pl.pallas_call(
        _proj_body,
        grid=(N_NODES // NODE_BLK,),
        in_specs=[pl.BlockSpec((NODE_BLK, D), lambda i: (i, 0)),
                  pl.BlockSpec((D, D), lambda i: (0, 0)),
                  pl.BlockSpec((D, D), lambda i: (0, 0)),
                  pl.BlockSpec((1, D), lambda i: (0, 0))],
        out_specs=[pl.BlockSpec((NODE_BLK, D), lambda i: (i, 0)),
                   pl.BlockSpec((NODE_BLK, D), lambda i: (i, 0))],
        out_shape=[jax.ShapeDtypeStruct((N_NODES, D), jnp.float32),
                   jax.ShapeDtypeStruct((N_NODES, D), jnp.float32)],
    )(x, ws, wd, be.reshape(1, D))


def _edge0_body(ea_ref, w_ref, pe_ref):
    pe_ref[...] = jnp.dot(ea_ref[...], w_ref[...],
                          preferred_element_type=jnp.float32)


def _edge0(ea, w):
    return pl.pallas_call(
        _edge0_body,
        grid=(N_EDGES // EDGE_BLK,),
        in_specs=[pl.BlockSpec((EDGE_BLK, D), lambda i: (i, 0)),
                  pl.BlockSpec((D, D), lambda i: (0, 0))],
        out_specs=pl.BlockSpec((EDGE_BLK, D), lambda i: (i, 0)),
        out_shape=jax.ShapeDtypeStruct((N_EDGES, D), jnp.float32),
    )(ea, w)


def _edge_body(ea_ref, eo_ref, w_ref, ea_out_ref, pe_ref):
    ea = ea_ref[...] + eo_ref[...]
    ea_out_ref[...] = ea
    pe_ref[...] = jnp.dot(ea, w_ref[...], preferred_element_type=jnp.float32)


def _edge(ea, eo, w):
    return pl.pallas_call(
        _edge_body,
        grid=(N_EDGES // EDGE_BLK,),
        in_specs=[pl.BlockSpec((EDGE_BLK, D), lambda i: (i, 0)),
                  pl.BlockSpec((EDGE_BLK, D), lambda i: (i, 0)),
                  pl.BlockSpec((D, D), lambda i: (0, 0))],
        out_specs=[pl.BlockSpec((EDGE_BLK, D), lambda i: (i, 0)),
                   pl.BlockSpec((EDGE_BLK, D), lambda i: (i, 0))],
        out_shape=[jax.ShapeDtypeStruct((N_EDGES, D), jnp.float32),
                   jax.ShapeDtypeStruct((N_EDGES, D), jnp.float32)],
    )(ea, eo, w)


def _resid_body(ea_ref, eo_ref, out_ref):
    out_ref[...] = ea_ref[...] + eo_ref[...]


def _resid(ea, eo):
    return pl.pallas_call(
        _resid_body,
        grid=(N_EDGES // EDGE_BLK,),
        in_specs=[pl.BlockSpec((EDGE_BLK, D), lambda i: (i, 0)),
                  pl.BlockSpec((EDGE_BLK, D), lambda i: (i, 0))],
        out_specs=pl.BlockSpec((EDGE_BLK, D), lambda i: (i, 0)),
        out_shape=jax.ShapeDtypeStruct((N_EDGES, D), jnp.float32),
    )(ea, eo)


def _node_body(x_ref, a0_ref, a1_ref, wx_ref, wa_ref, bn_ref, out_ref):
    x = x_ref[...]
    agg = a0_ref[...] + a1_ref[...]
    h = (jnp.dot(x, wx_ref[...], preferred_element_type=jnp.float32)
         + jnp.dot(agg, wa_ref[...], preferred_element_type=jnp.float32)
         + bn_ref[...])
    out_ref[...] = jnp.maximum(h, 0.0) + x


def _node(x, a0, a1, wx, wa, bn):
    return pl.pallas_call(
        _node_body,
        grid=(N_NODES // NODE_BLK,),
        in_specs=[pl.BlockSpec((NODE_BLK, D), lambda i: (i, 0)),
                  pl.BlockSpec((NODE_BLK, D), lambda i: (i, 0)),
                  pl.BlockSpec((NODE_BLK, D), lambda i: (i, 0)),
                  pl.BlockSpec((D, D), lambda i: (0, 0)),
                  pl.BlockSpec((D, D), lambda i: (0, 0)),
                  pl.BlockSpec((1, D), lambda i: (0, 0))],
        out_specs=pl.BlockSpec((NODE_BLK, D), lambda i: (i, 0)),
        out_shape=jax.ShapeDtypeStruct((N_NODES, D), jnp.float32),
    )(x, a0, a1, wx, wa, bn.reshape(1, D))


_MESH = plsc.VectorSubcoreMesh(core_axis_name="c", subcore_axis_name="s")

_SCRATCH = (
    [pltpu.VMEM((CHUNK,), jnp.int32) for _ in range(2 * NSETS)]
    + [pltpu.VMEM((CHUNK, D), jnp.float32) for _ in range(3 * NSETS)]
    + [pltpu.VMEM_SHARED((N_PAD, D), jnp.float32)]
    + [pltpu.SemaphoreType.DMA for _ in range(3 * NSETS)]
)


@functools.partial(
    pl.kernel,
    out_type=(jax.ShapeDtypeStruct((N_EDGES, D), jnp.float32),
              jax.ShapeDtypeStruct((NC, N_PAD, D), jnp.float32)),
    mesh=_MESH,
    scratch_types=_SCRATCH,
)
def _sc_edge(ps_hbm, pd_hbm, pe_hbm, src_hbm, dst_hbm, zero_hbm,
             eo_hbm, agg_hbm, *scr):
    svs = scr[0:NSETS]
    dvs = scr[NSETS:2 * NSETS]
    avs = scr[2 * NSETS:3 * NSETS]
    bvs = scr[3 * NSETS:4 * NSETS]
    cvs = scr[4 * NSETS:5 * NSETS]
    agg_sh = scr[5 * NSETS]
    isems = scr[5 * NSETS + 1:5 * NSETS + 1 + NSETS]
    gsems = scr[5 * NSETS + 1 + NSETS:5 * NSETS + 1 + 2 * NSETS]
    osems = scr[5 * NSETS + 1 + 2 * NSETS:5 * NSETS + 1 + 3 * NSETS]

    cid = lax.axis_index("c")
    sid = lax.axis_index("s")
    wid = sid * NC + cid
    base = wid * EPW
    L = EPW

    row0 = sid * ROWS_PER_SUB
    pltpu.sync_copy(zero_hbm.at[pl.ds(row0, ROWS_PER_SUB)],
                    agg_sh.at[pl.ds(row0, ROWS_PER_SUB)])
    plsc.subcore_barrier()

    def stage1(lo, s):
        pltpu.async_copy(src_hbm.at[pl.ds(base + lo, CHUNK)], svs[s], isems[s])
        pltpu.async_copy(dst_hbm.at[pl.ds(base + lo, CHUNK)], dvs[s], isems[s])

    def drain_stores(s):
        pltpu.make_async_copy(cvs[s], eo_hbm.at[pl.ds(base, CHUNK)],
                              osems[s]).wait()

    def stage2(lo, s, drain):
        pltpu.make_async_copy(src_hbm.at[pl.ds(base, CHUNK)], svs[s],
                              isems[s]).wait()
        pltpu.make_async_copy(dst_hbm.at[pl.ds(base, CHUNK)], dvs[s],
                              isems[s]).wait()
        if drain is True:
            drain_stores(s)
        elif drain is not False:
            @pl.when(drain)
            def _():
                drain_stores(s)
        pltpu.async_copy(ps_hbm.at[svs[s]], avs[s], gsems[s])
        pltpu.async_copy(pd_hbm.at[dvs[s]], bvs[s], gsems[s])
        pltpu.async_copy(pe_hbm.at[pl.ds(base + lo, CHUNK)], cvs[s], gsems[s])

    def process(lo, s):
        pltpu.make_async_copy(ps_hbm.at[svs[s]], avs[s], gsems[s]).wait()
        pltpu.make_async_copy(pd_hbm.at[dvs[s]], bvs[s], gsems[s]).wait()
        pltpu.make_async_copy(pe_hbm.at[pl.ds(base, CHUNK)], cvs[s],
                              gsems[s]).wait()
        av, bv, cv = avs[s], bvs[s], cvs[s]

        @pl.loop(0, CHUNK)
        def _(r):
            for g in range(D // 16):
                sl = pl.ds(g * 16, 16)
                cv[r, sl] = jnp.maximum(av[r, sl] + bv[r, sl] + cv[r, sl], 0.0)
        pltpu.sync_copy(cv, agg_sh.at[dvs[s]], add=True)
        pltpu.async_copy(cv, eo_hbm.at[pl.ds(base + lo, CHUNK)], osems[s])

    for s in range(NSETS):
        stage1(s * CHUNK, s)
    for s in range(3):
        stage2(s * CHUNK, s, False)

    @pl.loop(0, L, step=RC)
    def _(lo):
        for j in range(NSETS):
            k = lo + j * CHUNK

            @pl.when(k < L)
            def _():
                process(k, j)

            @pl.when(k + RC < L)
            def _():
                stage1(k + RC, j)

            s2 = (j + 3) % NSETS

            @pl.when(k + 3 * CHUNK < L)
            def _():
                stage2(k + 3 * CHUNK, s2, k + 3 * CHUNK >= RC)

    for s in range(NSETS):
        drain_stores(s)

    plsc.subcore_barrier()
    pltpu.sync_copy(agg_sh.at[pl.ds(row0, ROWS_PER_SUB)],
                    agg_hbm.at[cid, pl.ds(row0, ROWS_PER_SUB)])


def kernel(x, edge_index, edge_attr, We, be, Wn, bn):
    src = edge_index[0].astype(jnp.int32)
    dst = edge_index[1].astype(jnp.int32)
    zero = jnp.zeros((N_PAD, D), jnp.float32)
    ea = edge_attr
    eo = None
    for i in range(M_LAYERS):
        ps, pd = _proj(x, We[i, :D], We[i, D:2 * D], be[i])
        if i == 0:
            pe = _edge0(ea, We[i, 2 * D:])
        else:
            ea, pe = _edge(ea, eo, We[i, 2 * D:])
        eo, agg = _sc_edge(ps, pd, pe, src, dst, zero)
        x = _node(x, agg[0, :N_NODES], agg[1, :N_NODES],
                  Wn[i, :D], Wn[i, D:], bn[i])
    ea = _resid(ea, eo)
    return x, ea
